# baseline (speedup 1.0000x reference)
import functools

import jax
import jax.numpy as jnp
import numpy as np
from jax import lax
from jax.experimental import pallas as pl
from jax.experimental.pallas import tpu as pltpu
from jax.experimental.pallas import tpu_sc as plsc

N = 10000
NP = 10240
E = 320000
D = 128
NC = 2
NS = 16
NW = NC * NS
CH = 64
NCHUNK = 160
EPT = NCHUNK * CH
EPAD = NW * EPT
NBUF = 5
NF = 2
NQ = 10
ROWS_PER_TILE = NP // NS

_PI = np.empty((D,), np.int32)
for _j in range(D // 32):
    for _k in range(16):
        _PI[32 * _j + _k] = 32 * _j + 2 * _k
        _PI[32 * _j + 16 + _k] = 32 * _j + 2 * _k + 1


def _scale_chunk(rbf, rf, wq_q):
    def group_body(g, carry):
        wv = wq_q[pl.ds(g * 16, 16)]
        ws = [jnp.full((16,), 0.0, jnp.float32) + wv[e16]
              for e16 in range(16)]

        def j_body(j, c2):
            lo = pl.ds(j * 32, 32)
            for e16 in range(16):
                e = g * 16 + e16
                x = plsc.bitcast(rbf[e, lo], jnp.int32)
                a = plsc.bitcast(lax.shift_left(x, 16), jnp.float32)
                b = plsc.bitcast(
                    lax.bitwise_and(x, jnp.int32(-65536)), jnp.float32)
                rf[e, pl.ds(j * 32, 16)] = a * ws[e16]
                rf[e, pl.ds(j * 32 + 16, 16)] = b * ws[e16]
            return c2

        lax.fori_loop(0, D // 32, j_body, 0)
        return carry

    lax.fori_loop(0, CH // 16, group_body, 0)


def _sc_agg_body(h_hbm, pack_hbm, wpack_hbm, zeros_hbm, out_hbm,
                 r0, r1, r2, r3, r4, f0, f1,
                 p0, p1, p2, p3, p4, p5, p6, p7, p8, p9,
                 w0, w1, w2, w3, w4, w5, w6, w7, w8, w9, agg,
                 g0, g1, g2, g3, g4, s0, s1,
                 i0, i1, i2, i3, i4, i5, i6, i7, i8, i9):
    rbf = (r0, r1, r2, r3, r4)
    rf = (f0, f1)
    pk = (p0, p1, p2, p3, p4, p5, p6, p7, p8, p9)
    wq = (w0, w1, w2, w3, w4, w5, w6, w7, w8, w9)
    gsem = (g0, g1, g2, g3, g4)
    ssem = (s0, s1)
    isem = (i0, i1, i2, i3, i4, i5, i6, i7, i8, i9)
    cid = lax.axis_index("c")
    sid = lax.axis_index("s")
    wid = cid * NS + sid

    stripe = pl.ds(sid * ROWS_PER_TILE, ROWS_PER_TILE)
    pltpu.sync_copy(zeros_hbm.at[stripe], agg.at[stripe])

    for q in range(6):
        pltpu.async_copy(pack_hbm.at[wid, q], pk[q], isem[q])
        pltpu.async_copy(wpack_hbm.at[wid, q], wq[q], isem[q])
    plsc.subcore_barrier()
    for c0 in range(4):
        pltpu.make_async_copy(pack_hbm.at[wid, c0], pk[c0],
                              isem[c0]).wait()
        pltpu.make_async_copy(wpack_hbm.at[wid, c0], wq[c0],
                              isem[c0]).wait()
        pltpu.async_copy(h_hbm.at[pk[c0].at[0]], rbf[c0], gsem[c0])

    def dec_body(t, carry):
        for i in range(NQ):
            c = NQ * t + i
            b = i % NBUF
            f = i % NF
            q = i
            pltpu.make_async_copy(h_hbm.at[pk[q].at[0]], rbf[b],
                                  gsem[b]).wait()

            @pl.when(c >= NF)
            def _():
                qm = (i - NF) % NQ
                pltpu.make_async_copy(rf[f], agg.at[pk[qm].at[1]],
                                      ssem[f]).wait()

            _scale_chunk(rbf[b], rf[f], wq[q])
            pltpu.async_copy(rf[f], agg.at[pk[q].at[1]], ssem[f],
                             add=True)

            @pl.when(c + 4 < NCHUNK)
            def _():
                b4 = (i + 4) % NBUF
                q4 = (i + 4) % NQ
                pltpu.make_async_copy(pack_hbm.at[wid, c + 4], pk[q4],
                                      isem[q4]).wait()
                pltpu.make_async_copy(wpack_hbm.at[wid, c + 4], wq[q4],
                                      isem[q4]).wait()
                pltpu.async_copy(h_hbm.at[pk[q4].at[0]], rbf[b4],
                                 gsem[b4])

            @pl.when(c + 6 < NCHUNK)
            def _():
                q6 = (i + 6) % NQ
                pltpu.async_copy(pack_hbm.at[wid, c + 6], pk[q6],
                                 isem[q6])
                pltpu.async_copy(wpack_hbm.at[wid, c + 6], wq[q6],
                                 isem[q6])

        return carry

    lax.fori_loop(0, NCHUNK // NQ, dec_body, 0)

    for j in range(NCHUNK - NF, NCHUNK):
        pltpu.make_async_copy(rf[j % NF],
                              agg.at[pk[j % NQ].at[1]],
                              ssem[j % NF]).wait()

    plsc.subcore_barrier()
    pltpu.sync_copy(agg.at[stripe], out_hbm.at[cid, stripe])


def _sc_aggregate(hbf, pack, wpack, zeros):
    mesh = plsc.VectorSubcoreMesh(core_axis_name="c", subcore_axis_name="s")
    f = pl.kernel(
        _sc_agg_body,
        out_type=jax.ShapeDtypeStruct((NC, NP, D), jnp.float32),
        mesh=mesh,
        compiler_params=pltpu.CompilerParams(
            use_tc_tiling_on_sc=False, needs_layout_passes=False),
        scratch_types=(
            [pltpu.VMEM((CH, D), jnp.bfloat16)] * NBUF
            + [pltpu.VMEM((CH, D), jnp.float32)] * NF
            + [pltpu.VMEM((2, CH), jnp.int32)] * NQ
            + [pltpu.VMEM((CH,), jnp.float32)] * NQ
            + [pltpu.VMEM_SHARED((NP, D), jnp.float32)]
            + [pltpu.SemaphoreType.DMA] * (NBUF + NF + NQ)
        ),
    )
    return f(hbf, pack, wpack, zeros)


def _tc_conv_body(p_ref, wt_ref, b_ref, o_ref):
    acc = p_ref[0] + p_ref[1]
    h = jnp.dot(acc, wt_ref[...], preferred_element_type=jnp.float32)
    h = jnp.maximum(h + b_ref[...], 0.0)
    o_ref[...] = h.astype(jnp.bfloat16)


def _tc_conv(p, wt, b2d):
    blk = 1024
    return pl.pallas_call(
        _tc_conv_body,
        grid=(NP // blk,),
        in_specs=[
            pl.BlockSpec((NC, blk, D), lambda i: (0, i, 0)),
            pl.BlockSpec((D, D), lambda i: (0, 0)),
            pl.BlockSpec((1, D), lambda i: (0, 0)),
        ],
        out_specs=pl.BlockSpec((blk, D), lambda i: (i, 0)),
        out_shape=jax.ShapeDtypeStruct((NP, D), jnp.bfloat16),
    )(p, wt, b2d)


def _tc_final_body(p_ref, w3t_ref, b3_ref, wp1t_ref, bp1_ref, wp2_ref,
                   bp2_ref, scores_ref, h_ref):
    acc = p_ref[0] + p_ref[1]
    h = jnp.dot(acc, w3t_ref[...], preferred_element_type=jnp.float32)
    h = h + b3_ref[...]
    h_ref[...] = h
    pooled = jnp.sum(h[:N], axis=0, keepdims=True) / N
    a = jnp.dot(pooled, wp1t_ref[...],
                preferred_element_type=jnp.float32) + bp1_ref[...]
    a = jnp.maximum(a, 0.0)
    s = jnp.sum(a * wp2_ref[...]) + bp2_ref[0, 0]
    s = 1.0 / (1.0 + jnp.exp(-s))
    scores_ref[...] = jnp.full((NP, 1), s, jnp.float32)


def _tc_final(p, w3t, b3_2d, wp1t, bp1_2d, wp2, bp2_2d):
    return pl.pallas_call(
        _tc_final_body,
        out_shape=(
            jax.ShapeDtypeStruct((NP, 1), jnp.float32),
            jax.ShapeDtypeStruct((NP, D), jnp.float32),
        ),
    )(p, w3t, b3_2d, wp1t, bp1_2d, wp2, bp2_2d)


def kernel(x, edge_index, edge_weight, W1, b1, W2, b2, W3, b3,
           Wp1, bp1, Wp2, bp2):
    npad = EPAD - E
    row3 = jnp.concatenate(
        [edge_index[0].astype(jnp.int32),
         jnp.zeros((npad,), jnp.int32)]).reshape(NW, NCHUNK, CH)
    col3 = jnp.concatenate(
        [edge_index[1].astype(jnp.int32),
         jnp.zeros((npad,), jnp.int32)]).reshape(NW, NCHUNK, CH)
    wpack = jnp.concatenate(
        [edge_weight, jnp.zeros((npad,), jnp.float32)]
    ).reshape(NW, NCHUNK, CH)
    pack = jnp.stack([row3, col3], axis=2)
    zeros = jnp.zeros((NP, D), jnp.float32)

    pi = jnp.asarray(_PI)
    w1tp = W1.T[pi]
    w2tp = W2.T[pi]
    w3tp = W3.T[pi]
    wp1t = Wp1.T

    xbf = jnp.concatenate(
        [x, jnp.zeros((NP - N, D), jnp.float32)]).astype(jnp.bfloat16)
    p = _sc_aggregate(xbf, pack, wpack, zeros)
    hbf = _tc_conv(p, w1tp, b1.reshape(1, D))
    p = _sc_aggregate(hbf, pack, wpack, zeros)
    hbf = _tc_conv(p, w2tp, b2.reshape(1, D))
    p = _sc_aggregate(hbf, pack, wpack, zeros)
    scores, hout = _tc_final(p, w3tp, b3.reshape(1, D), wp1t,
                             bp1.reshape(1, D // 2), Wp2,
                             bp2.reshape(1, 1))
    return (scores[:N], hout[:N])


# speedup vs baseline: 1.0018x; 1.0018x over previous
import functools

import jax
import jax.numpy as jnp
import numpy as np
from jax import lax
from jax.experimental import pallas as pl
from jax.experimental.pallas import tpu as pltpu
from jax.experimental.pallas import tpu_sc as plsc

N = 10000
NP = 10240
E = 320000
D = 128
NC = 2
NS = 16
NW = NC * NS
CH = 64
NCHUNK = 160
EPT = NCHUNK * CH
EPAD = NW * EPT
NBUF = 5
NF = 2
NQ = 10
ROWS_PER_TILE = NP // NS

_PI = np.empty((D,), np.int32)
for _j in range(D // 32):
    for _k in range(16):
        _PI[32 * _j + _k] = 32 * _j + 2 * _k
        _PI[32 * _j + 16 + _k] = 32 * _j + 2 * _k + 1


def _scale_chunk(rbf, rf, wq_q):
    def group_body(g, carry):
        wv = wq_q[pl.ds(g * 16, 16)]
        ws = [jnp.full((16,), 0.0, jnp.float32) + wv[e16]
              for e16 in range(16)]

        def j_body(j, c2):
            lo = pl.ds(j * 32, 32)
            for e16 in range(16):
                e = g * 16 + e16
                x = plsc.bitcast(rbf[e, lo], jnp.int32)
                a = plsc.bitcast(lax.shift_left(x, 16), jnp.float32)
                b = plsc.bitcast(
                    lax.bitwise_and(x, jnp.int32(-65536)), jnp.float32)
                rf[e, pl.ds(j * 32, 16)] = a * ws[e16]
                rf[e, pl.ds(j * 32 + 16, 16)] = b * ws[e16]
            return c2

        lax.fori_loop(0, D // 32, j_body, 0)
        return carry

    lax.fori_loop(0, CH // 16, group_body, 0)


def _sc_agg_body(h_hbm, pack_hbm, wpack_hbm, zeros_hbm, out_hbm,
                 r0, r1, r2, r3, r4, f0, f1,
                 p0, p1, p2, p3, p4, p5, p6, p7, p8, p9,
                 w0, w1, w2, w3, w4, w5, w6, w7, w8, w9, agg,
                 g0, g1, g2, g3, g4, s0, s1, zsem,
                 i0, i1, i2, i3, i4, i5, i6, i7, i8, i9):
    rbf = (r0, r1, r2, r3, r4)
    rf = (f0, f1)
    pk = (p0, p1, p2, p3, p4, p5, p6, p7, p8, p9)
    wq = (w0, w1, w2, w3, w4, w5, w6, w7, w8, w9)
    gsem = (g0, g1, g2, g3, g4)
    ssem = (s0, s1)
    isem = (i0, i1, i2, i3, i4, i5, i6, i7, i8, i9)
    cid = lax.axis_index("c")
    sid = lax.axis_index("s")
    wid = cid * NS + sid

    stripe = pl.ds(sid * ROWS_PER_TILE, ROWS_PER_TILE)
    pltpu.async_copy(zeros_hbm.at[stripe], agg.at[stripe], zsem)

    for q in range(6):
        pltpu.async_copy(pack_hbm.at[wid, q], pk[q], isem[q])
        pltpu.async_copy(wpack_hbm.at[wid, q], wq[q], isem[q])
    pltpu.make_async_copy(zeros_hbm.at[stripe], agg.at[stripe],
                          zsem).wait()
    plsc.subcore_barrier()
    for c0 in range(4):
        pltpu.make_async_copy(pack_hbm.at[wid, c0], pk[c0],
                              isem[c0]).wait()
        pltpu.make_async_copy(wpack_hbm.at[wid, c0], wq[c0],
                              isem[c0]).wait()
        pltpu.async_copy(h_hbm.at[pk[c0].at[0]], rbf[c0], gsem[c0])

    def dec_body(t, carry):
        for i in range(NQ):
            c = NQ * t + i
            b = i % NBUF
            f = i % NF
            q = i
            pltpu.make_async_copy(h_hbm.at[pk[q].at[0]], rbf[b],
                                  gsem[b]).wait()

            @pl.when(c >= NF)
            def _():
                qm = (i - NF) % NQ
                pltpu.make_async_copy(rf[f], agg.at[pk[qm].at[1]],
                                      ssem[f]).wait()

            _scale_chunk(rbf[b], rf[f], wq[q])
            pltpu.async_copy(rf[f], agg.at[pk[q].at[1]], ssem[f],
                             add=True)

            @pl.when(c + 4 < NCHUNK)
            def _():
                b4 = (i + 4) % NBUF
                q4 = (i + 4) % NQ
                pltpu.make_async_copy(pack_hbm.at[wid, c + 4], pk[q4],
                                      isem[q4]).wait()
                pltpu.make_async_copy(wpack_hbm.at[wid, c + 4], wq[q4],
                                      isem[q4]).wait()
                pltpu.async_copy(h_hbm.at[pk[q4].at[0]], rbf[b4],
                                 gsem[b4])

            @pl.when(c + 6 < NCHUNK)
            def _():
                q6 = (i + 6) % NQ
                pltpu.async_copy(pack_hbm.at[wid, c + 6], pk[q6],
                                 isem[q6])
                pltpu.async_copy(wpack_hbm.at[wid, c + 6], wq[q6],
                                 isem[q6])

        return carry

    lax.fori_loop(0, NCHUNK // NQ, dec_body, 0)

    for j in range(NCHUNK - NF, NCHUNK):
        pltpu.make_async_copy(rf[j % NF],
                              agg.at[pk[j % NQ].at[1]],
                              ssem[j % NF]).wait()

    plsc.subcore_barrier()
    pltpu.sync_copy(agg.at[stripe], out_hbm.at[cid, stripe])


def _sc_aggregate(hbf, pack, wpack, zeros):
    mesh = plsc.VectorSubcoreMesh(core_axis_name="c", subcore_axis_name="s")
    f = pl.kernel(
        _sc_agg_body,
        out_type=jax.ShapeDtypeStruct((NC, NP, D), jnp.float32),
        mesh=mesh,
        compiler_params=pltpu.CompilerParams(
            use_tc_tiling_on_sc=False, needs_layout_passes=False),
        scratch_types=(
            [pltpu.VMEM((CH, D), jnp.bfloat16)] * NBUF
            + [pltpu.VMEM((CH, D), jnp.float32)] * NF
            + [pltpu.VMEM((2, CH), jnp.int32)] * NQ
            + [pltpu.VMEM((CH,), jnp.float32)] * NQ
            + [pltpu.VMEM_SHARED((NP, D), jnp.float32)]
            + [pltpu.SemaphoreType.DMA] * (NBUF + NF + 1 + NQ)
        ),
    )
    return f(hbf, pack, wpack, zeros)


def _tc_conv_body(p_ref, wt_ref, b_ref, o_ref):
    acc = p_ref[0] + p_ref[1]
    h = jnp.dot(acc, wt_ref[...], preferred_element_type=jnp.float32)
    h = jnp.maximum(h + b_ref[...], 0.0)
    o_ref[...] = h.astype(jnp.bfloat16)


def _tc_conv(p, wt, b2d):
    blk = 1024
    return pl.pallas_call(
        _tc_conv_body,
        grid=(NP // blk,),
        in_specs=[
            pl.BlockSpec((NC, blk, D), lambda i: (0, i, 0)),
            pl.BlockSpec((D, D), lambda i: (0, 0)),
            pl.BlockSpec((1, D), lambda i: (0, 0)),
        ],
        out_specs=pl.BlockSpec((blk, D), lambda i: (i, 0)),
        out_shape=jax.ShapeDtypeStruct((NP, D), jnp.bfloat16),
    )(p, wt, b2d)


def _tc_final_body(p_ref, w3t_ref, b3_ref, wp1t_ref, bp1_ref, wp2_ref,
                   bp2_ref, scores_ref, h_ref):
    acc = p_ref[0] + p_ref[1]
    h = jnp.dot(acc, w3t_ref[...], preferred_element_type=jnp.float32)
    h = h + b3_ref[...]
    h_ref[...] = h
    pooled = jnp.sum(h[:N], axis=0, keepdims=True) / N
    a = jnp.dot(pooled, wp1t_ref[...],
                preferred_element_type=jnp.float32) + bp1_ref[...]
    a = jnp.maximum(a, 0.0)
    s = jnp.sum(a * wp2_ref[...]) + bp2_ref[0, 0]
    s = 1.0 / (1.0 + jnp.exp(-s))
    scores_ref[...] = jnp.full((NP, 1), s, jnp.float32)


def _tc_final(p, w3t, b3_2d, wp1t, bp1_2d, wp2, bp2_2d):
    return pl.pallas_call(
        _tc_final_body,
        out_shape=(
            jax.ShapeDtypeStruct((NP, 1), jnp.float32),
            jax.ShapeDtypeStruct((NP, D), jnp.float32),
        ),
    )(p, w3t, b3_2d, wp1t, bp1_2d, wp2, bp2_2d)


def kernel(x, edge_index, edge_weight, W1, b1, W2, b2, W3, b3,
           Wp1, bp1, Wp2, bp2):
    npad = EPAD - E
    row3 = jnp.concatenate(
        [edge_index[0].astype(jnp.int32),
         jnp.zeros((npad,), jnp.int32)]).reshape(NW, NCHUNK, CH)
    col3 = jnp.concatenate(
        [edge_index[1].astype(jnp.int32),
         jnp.zeros((npad,), jnp.int32)]).reshape(NW, NCHUNK, CH)
    wpack = jnp.concatenate(
        [edge_weight, jnp.zeros((npad,), jnp.float32)]
    ).reshape(NW, NCHUNK, CH)
    pack = jnp.stack([row3, col3], axis=2)
    zeros = jnp.zeros((NP, D), jnp.float32)

    pi = jnp.asarray(_PI)
    w1tp = W1.T[pi]
    w2tp = W2.T[pi]
    w3tp = W3.T[pi]
    wp1t = Wp1.T

    xbf = jnp.concatenate(
        [x, jnp.zeros((NP - N, D), jnp.float32)]).astype(jnp.bfloat16)
    p = _sc_aggregate(xbf, pack, wpack, zeros)
    hbf = _tc_conv(p, w1tp, b1.reshape(1, D))
    p = _sc_aggregate(hbf, pack, wpack, zeros)
    hbf = _tc_conv(p, w2tp, b2.reshape(1, D))
    p = _sc_aggregate(hbf, pack, wpack, zeros)
    scores, hout = _tc_final(p, w3tp, b3.reshape(1, D), wp1t,
                             bp1.reshape(1, D // 2), Wp2,
                             bp2.reshape(1, 1))
    return (scores[:N], hout[:N])


# speedup vs baseline: 1.0311x; 1.0292x over previous
import functools

import jax
import jax.numpy as jnp
import numpy as np
from jax import lax
from jax.experimental import pallas as pl
from jax.experimental.pallas import tpu as pltpu
from jax.experimental.pallas import tpu_sc as plsc

N = 10000
NP = 10240
E = 320000
D = 128
NC = 2
NS = 16
NW = NC * NS
CH = 64
NCHUNK = 160
EPT = NCHUNK * CH
EPAD = NW * EPT
NBUF = 5
NF = 2
NQ = 10
ROWS_PER_TILE = NP // NS

_PI = np.empty((D,), np.int32)
for _j in range(D // 32):
    for _k in range(16):
        _PI[32 * _j + _k] = 32 * _j + 2 * _k
        _PI[32 * _j + 16 + _k] = 32 * _j + 2 * _k + 1


def _scale_chunk(rbf, rf, wq_q):
    def group_body(g, carry):
        wv = wq_q[pl.ds(g * 16, 16)]
        ws = [jnp.full((16,), 0.0, jnp.float32) + wv[e16]
              for e16 in range(16)]

        def j_body(j, c2):
            lo = pl.ds(j * 32, 32)
            for e16 in range(16):
                e = g * 16 + e16
                x = plsc.bitcast(rbf[e, lo], jnp.int32)
                a = plsc.bitcast(lax.shift_left(x, 16), jnp.float32)
                b = plsc.bitcast(
                    lax.bitwise_and(x, jnp.int32(-65536)), jnp.float32)
                rf[e, pl.ds(j * 32, 16)] = a * ws[e16]
                rf[e, pl.ds(j * 32 + 16, 16)] = b * ws[e16]
            return c2

        lax.fori_loop(0, D // 32, j_body, 0)
        return carry

    lax.fori_loop(0, CH // 16, group_body, 0)


def _sc_agg_body(h_hbm, pack_hbm, wpack_hbm, zeros_hbm, out_hbm,
                 r0, r1, r2, r3, r4, f0, f1,
                 p0, p1, p2, p3, p4, p5, p6, p7, p8, p9,
                 w0, w1, w2, w3, w4, w5, w6, w7, w8, w9, agg,
                 g0, g1, g2, g3, g4, s0, s1, zsem,
                 i0, i1, i2, i3, i4, i5, i6, i7, i8, i9):
    rbf = (r0, r1, r2, r3, r4)
    rf = (f0, f1)
    pk = (p0, p1, p2, p3, p4, p5, p6, p7, p8, p9)
    wq = (w0, w1, w2, w3, w4, w5, w6, w7, w8, w9)
    gsem = (g0, g1, g2, g3, g4)
    ssem = (s0, s1)
    isem = (i0, i1, i2, i3, i4, i5, i6, i7, i8, i9)
    cid = lax.axis_index("c")
    sid = lax.axis_index("s")
    wid = cid * NS + sid

    stripe = pl.ds(sid * ROWS_PER_TILE, ROWS_PER_TILE)
    pltpu.async_copy(zeros_hbm.at[stripe], agg.at[stripe], zsem)

    for q in range(6):
        pltpu.async_copy(pack_hbm.at[wid, q], pk[q], isem[q])
        pltpu.async_copy(wpack_hbm.at[wid, q], wq[q], isem[q])
    pltpu.make_async_copy(zeros_hbm.at[stripe], agg.at[stripe],
                          zsem).wait()
    plsc.subcore_barrier()
    for c0 in range(4):
        pltpu.make_async_copy(pack_hbm.at[wid, c0], pk[c0],
                              isem[c0]).wait()
        pltpu.make_async_copy(wpack_hbm.at[wid, c0], wq[c0],
                              isem[c0]).wait()
        pltpu.async_copy(h_hbm.at[pk[c0].at[0]], rbf[c0], gsem[c0])

    def dec_body(t, carry):
        for i in range(NQ):
            c = NQ * t + i
            b = i % NBUF
            f = i % NF
            q = i
            pltpu.make_async_copy(h_hbm.at[pk[q].at[0]], rbf[b],
                                  gsem[b]).wait()

            @pl.when(c >= NF)
            def _():
                qm = (i - NF) % NQ
                pltpu.make_async_copy(rf[f], agg.at[pk[qm].at[1]],
                                      ssem[f]).wait()

            _scale_chunk(rbf[b], rf[f], wq[q])
            pltpu.async_copy(rf[f], agg.at[pk[q].at[1]], ssem[f],
                             add=True)

            @pl.when(c + 4 < NCHUNK)
            def _():
                b4 = (i + 4) % NBUF
                q4 = (i + 4) % NQ
                pltpu.make_async_copy(pack_hbm.at[wid, c + 4], pk[q4],
                                      isem[q4]).wait()
                pltpu.make_async_copy(wpack_hbm.at[wid, c + 4], wq[q4],
                                      isem[q4]).wait()
                pltpu.async_copy(h_hbm.at[pk[q4].at[0]], rbf[b4],
                                 gsem[b4])

            @pl.when(c + 6 < NCHUNK)
            def _():
                q6 = (i + 6) % NQ
                pltpu.async_copy(pack_hbm.at[wid, c + 6], pk[q6],
                                 isem[q6])
                pltpu.async_copy(wpack_hbm.at[wid, c + 6], wq[q6],
                                 isem[q6])

        return carry

    lax.fori_loop(0, NCHUNK // NQ, dec_body, 0)

    for j in range(NCHUNK - NF, NCHUNK):
        pltpu.make_async_copy(rf[j % NF],
                              agg.at[pk[j % NQ].at[1]],
                              ssem[j % NF]).wait()

    plsc.subcore_barrier()
    pltpu.sync_copy(agg.at[stripe], out_hbm.at[cid, stripe])


def _sc_aggregate(hbf, pack, wpack, zeros):
    mesh = plsc.VectorSubcoreMesh(core_axis_name="c", subcore_axis_name="s")
    f = pl.kernel(
        _sc_agg_body,
        out_type=jax.ShapeDtypeStruct((NC, NP, D), jnp.float32),
        mesh=mesh,
        compiler_params=pltpu.CompilerParams(
            use_tc_tiling_on_sc=False, needs_layout_passes=False),
        scratch_types=(
            [pltpu.VMEM((CH, D), jnp.bfloat16)] * NBUF
            + [pltpu.VMEM((CH, D), jnp.float32)] * NF
            + [pltpu.VMEM((2, CH), jnp.int32)] * NQ
            + [pltpu.VMEM((CH,), jnp.float32)] * NQ
            + [pltpu.VMEM_SHARED((NP, D), jnp.float32)]
            + [pltpu.SemaphoreType.DMA] * (NBUF + NF + 1 + NQ)
        ),
    )
    return f(hbf, pack, wpack, zeros)


def _tc_conv_body(p_ref, wt_ref, b_ref, o_ref):
    acc = p_ref[0] + p_ref[1]
    h = jnp.dot(acc, wt_ref[...], preferred_element_type=jnp.float32)
    h = jnp.maximum(h + b_ref[...], 0.0)
    o_ref[...] = h.astype(jnp.bfloat16)


def _tc_conv(p, wt, b2d):
    blk = 1024
    return pl.pallas_call(
        _tc_conv_body,
        grid=(NP // blk,),
        in_specs=[
            pl.BlockSpec((NC, blk, D), lambda i: (0, i, 0)),
            pl.BlockSpec((D, D), lambda i: (0, 0)),
            pl.BlockSpec((1, D), lambda i: (0, 0)),
        ],
        out_specs=pl.BlockSpec((blk, D), lambda i: (i, 0)),
        out_shape=jax.ShapeDtypeStruct((NP, D), jnp.bfloat16),
    )(p, wt, b2d)


def _tc_final_body(p_ref, w3t_ref, b3_ref, wp1t_ref, bp1_ref, wp2_ref,
                   bp2_ref, scores_ref, h_ref, acc_ref):
    i = pl.program_id(0)
    nblk = pl.num_programs(0)
    acc = p_ref[0] + p_ref[1]
    h = jnp.dot(acc, w3t_ref[...], preferred_element_type=jnp.float32)
    h = h + b3_ref[...]
    h_ref[...] = h

    @pl.when(i == 0)
    def _():
        acc_ref[...] = jnp.zeros_like(acc_ref)

    blk = h.shape[0]
    base = i * blk
    ridx = jax.lax.broadcasted_iota(jnp.int32, h.shape, 0) + base
    acc_ref[...] += jnp.sum(jnp.where(ridx < N, h, 0.0), axis=0,
                            keepdims=True)

    @pl.when(i == nblk - 1)
    def _():
        pooled = acc_ref[...] / N
        a = jnp.dot(pooled, wp1t_ref[...],
                    preferred_element_type=jnp.float32) + bp1_ref[...]
        a = jnp.maximum(a, 0.0)
        s = jnp.sum(a * wp2_ref[...]) + bp2_ref[0, 0]
        s = 1.0 / (1.0 + jnp.exp(-s))
        scores_ref[...] = jnp.full((NP, 1), s, jnp.float32)


def _tc_final(p, w3t, b3_2d, wp1t, bp1_2d, wp2, bp2_2d):
    blk = 1024
    return pl.pallas_call(
        _tc_final_body,
        grid=(NP // blk,),
        in_specs=[
            pl.BlockSpec((NC, blk, D), lambda i: (0, i, 0)),
            pl.BlockSpec((D, D), lambda i: (0, 0)),
            pl.BlockSpec((1, D), lambda i: (0, 0)),
            pl.BlockSpec((D, D // 2), lambda i: (0, 0)),
            pl.BlockSpec((1, D // 2), lambda i: (0, 0)),
            pl.BlockSpec((1, D // 2), lambda i: (0, 0)),
            pl.BlockSpec((1, 1), lambda i: (0, 0)),
        ],
        out_specs=(
            pl.BlockSpec((NP, 1), lambda i: (0, 0)),
            pl.BlockSpec((blk, D), lambda i: (i, 0)),
        ),
        out_shape=(
            jax.ShapeDtypeStruct((NP, 1), jnp.float32),
            jax.ShapeDtypeStruct((NP, D), jnp.float32),
        ),
        scratch_shapes=[pltpu.VMEM((1, D), jnp.float32)],
    )(p, w3t, b3_2d, wp1t, bp1_2d, wp2, bp2_2d)


def kernel(x, edge_index, edge_weight, W1, b1, W2, b2, W3, b3,
           Wp1, bp1, Wp2, bp2):
    npad = EPAD - E
    row3 = jnp.concatenate(
        [edge_index[0].astype(jnp.int32),
         jnp.zeros((npad,), jnp.int32)]).reshape(NW, NCHUNK, CH)
    col3 = jnp.concatenate(
        [edge_index[1].astype(jnp.int32),
         jnp.zeros((npad,), jnp.int32)]).reshape(NW, NCHUNK, CH)
    wpack = jnp.concatenate(
        [edge_weight, jnp.zeros((npad,), jnp.float32)]
    ).reshape(NW, NCHUNK, CH)
    pack = jnp.stack([row3, col3], axis=2)
    zeros = jnp.zeros((NP, D), jnp.float32)

    pi = jnp.asarray(_PI)
    w1tp = W1.T[pi]
    w2tp = W2.T[pi]
    w3tp = W3.T[pi]
    wp1t = Wp1.T

    xbf = jnp.concatenate(
        [x, jnp.zeros((NP - N, D), jnp.float32)]).astype(jnp.bfloat16)
    p = _sc_aggregate(xbf, pack, wpack, zeros)
    hbf = _tc_conv(p, w1tp, b1.reshape(1, D))
    p = _sc_aggregate(hbf, pack, wpack, zeros)
    hbf = _tc_conv(p, w2tp, b2.reshape(1, D))
    p = _sc_aggregate(hbf, pack, wpack, zeros)
    scores, hout = _tc_final(p, w3tp, b3.reshape(1, D), wp1t,
                             bp1.reshape(1, D // 2), Wp2,
                             bp2.reshape(1, 1))
    return (scores[:N], hout[:N])
